# initial kernel scaffold (unmeasured)
import jax
import jax.numpy as jnp
from jax import lax
from jax.experimental import pallas as pl
from jax.experimental.pallas import tpu as pltpu

N = 4
M = 4096
D = 4096
CH = M // N
TR = 256
EPS = 1e-6
F32 = jnp.float32


def kernel(partial, resid, gamma):
    gamma2 = gamma.reshape(1, D)

    def body(x_hbm, resid_hbm, gamma_ref, out_hbm,
             pbuf, send_buf, rs_recv, ag_recv,
             send_sems, recv_sems, copy_sems, credit_sem):
        xi = lax.axis_index("x")
        yi = lax.axis_index("y")
        zi = lax.axis_index("z")
        right = (xi, yi, (zi + 1) % N)
        left = (xi, yi, (zi - 1) % N)

        bar = pltpu.get_barrier_semaphore()
        pl.semaphore_signal(bar, 1, device_id=left)
        pl.semaphore_signal(bar, 1, device_id=right)
        pl.semaphore_wait(bar, 2)

        def rows(c):
            return pl.ds(c * CH, CH)

        def load(src, dst, sem):
            cp = pltpu.make_async_copy(src, dst, sem)
            cp.start()
            cp.wait()

        load(x_hbm.at[0, rows(zi), :], send_buf, copy_sems.at[0])
        load(x_hbm.at[0, rows((zi - 1) % N), :], pbuf, copy_sems.at[1])

        for s in range(N - 1):
            slot = s % 2
            rdma = pltpu.make_async_remote_copy(
                src_ref=send_buf,
                dst_ref=rs_recv.at[slot],
                send_sem=send_sems.at[s],
                recv_sem=recv_sems.at[s],
                device_id=right,
            )
            if s == 2:
                pl.semaphore_wait(credit_sem, 1)
            rdma.start()
            rdma.wait()
            for t in range(CH // TR):
                ts = pl.ds(t * TR, TR)
                send_buf[ts, :] = rs_recv[slot, ts, :] + pbuf[ts, :]
            if s == 0:
                pl.semaphore_signal(credit_sem, 1, device_id=left)
            if s < 2:
                load(x_hbm.at[0, rows((zi - s - 2) % N), :], pbuf,
                     copy_sems.at[(s + 1) % 2])

        own = (zi + 1) % N
        load(resid_hbm.at[rows(own), :], pbuf, copy_sems.at[0])
        for t in range(CH // TR):
            ts = pl.ds(t * TR, TR)
            y = send_buf[ts, :] + pbuf[ts, :]
            ms = jnp.mean(y * y, axis=-1, keepdims=True)
            send_buf[ts, :] = (y * lax.rsqrt(ms + EPS)) * gamma_ref[...]
        out_cp = pltpu.make_async_copy(
            send_buf, out_hbm.at[rows(own), :], copy_sems.at[2])
        out_cp.start()

        srcs = [send_buf, ag_recv.at[0], ag_recv.at[1]]
        for s in range(N - 1):
            slot = s % 2
            rdma = pltpu.make_async_remote_copy(
                src_ref=srcs[s],
                dst_ref=ag_recv.at[slot],
                send_sem=send_sems.at[3 + s],
                recv_sem=recv_sems.at[3 + s],
                device_id=right,
            )
            if s == 2:
                pl.semaphore_wait(credit_sem, 1)
            rdma.start()
            rdma.wait()
            if s == 1:
                pl.semaphore_signal(credit_sem, 1, device_id=left)
            origin = (zi - s) % N
            load(ag_recv.at[slot], out_hbm.at[rows(origin), :],
                 copy_sems.at[3])
        out_cp.wait()

    return pl.pallas_call(
        body,
        out_shape=jax.ShapeDtypeStruct((M, D), F32),
        in_specs=[
            pl.BlockSpec(memory_space=pltpu.MemorySpace.HBM),
            pl.BlockSpec(memory_space=pltpu.MemorySpace.HBM),
            pl.BlockSpec(memory_space=pltpu.MemorySpace.VMEM),
        ],
        out_specs=pl.BlockSpec(memory_space=pltpu.MemorySpace.HBM),
        scratch_shapes=[
            pltpu.VMEM((CH, D), F32),
            pltpu.VMEM((CH, D), F32),
            pltpu.VMEM((2, CH, D), F32),
            pltpu.VMEM((2, CH, D), F32),
            pltpu.SemaphoreType.DMA((6,)),
            pltpu.SemaphoreType.DMA((6,)),
            pltpu.SemaphoreType.DMA((4,)),
            pltpu.SemaphoreType.REGULAR,
        ],
        compiler_params=pltpu.CompilerParams(collective_id=0),
    )(partial, resid, gamma2)


# baseline (device time: 1214346 ns/iter reference)
import jax
import jax.numpy as jnp
from jax import lax
from jax.experimental import pallas as pl
from jax.experimental.pallas import tpu as pltpu

N = 4
M = 4096
D = 4096
ROUNDS = 2
RH = M // ROUNDS
CH = RH // N
TR = 128
EPS = 1e-6
F32 = jnp.float32


def kernel(partial, resid, gamma):
    gamma2 = gamma.reshape(1, D)

    def body(x_hbm, resid_hbm, gamma_ref, out_hbm,
             pbuf, send_buf, rs_recv, ag_recv,
             send_sems, recv_sems, copy_sems, credit_sem):
        xi = lax.axis_index("x")
        yi = lax.axis_index("y")
        zi = lax.axis_index("z")
        right = (xi, yi, (zi + 1) % N)
        left = (xi, yi, (zi - 1) % N)

        bar = pltpu.get_barrier_semaphore()
        pl.semaphore_signal(bar, 1, device_id=left)
        pl.semaphore_signal(bar, 1, device_id=right)
        pl.semaphore_wait(bar, 2)

        def load(src, dst, sem):
            cp = pltpu.make_async_copy(src, dst, sem)
            cp.start()
            cp.wait()

        def tiled(f):
            lax.fori_loop(0, CH // TR, lambda t, _: (f(pl.ds(t * TR, TR)), 0)[1], 0)

        def one_round(base):
            def rows(c):
                return pl.ds(base + c * CH, CH)

            load(x_hbm.at[0, rows(zi), :], send_buf, copy_sems.at[0])
            load(x_hbm.at[0, rows((zi - 1) % N), :], pbuf, copy_sems.at[1])

            for s in range(N - 1):
                slot = s % 2
                rdma = pltpu.make_async_remote_copy(
                    src_ref=send_buf,
                    dst_ref=rs_recv.at[slot],
                    send_sem=send_sems.at[s],
                    recv_sem=recv_sems.at[s],
                    device_id=right,
                )
                if s == 2:
                    pl.semaphore_wait(credit_sem, 1)
                rdma.start()
                rdma.wait()

                def add(ts, slot=slot):
                    send_buf[ts, :] = rs_recv[slot, ts, :] + pbuf[ts, :]
                tiled(add)
                if s == 0:
                    pl.semaphore_signal(credit_sem, 1, device_id=left)
                if s < 2:
                    load(x_hbm.at[0, rows((zi - s - 2) % N), :], pbuf,
                         copy_sems.at[(s + 1) % 2])

            own = (zi + 1) % N
            load(resid_hbm.at[rows(own), :], pbuf, copy_sems.at[0])

            def norm(ts):
                y = send_buf[ts, :] + pbuf[ts, :]
                ms = jnp.mean(y * y, axis=-1, keepdims=True)
                send_buf[ts, :] = (y * lax.rsqrt(ms + EPS)) * gamma_ref[...]
            tiled(norm)
            out_cp = pltpu.make_async_copy(
                send_buf, out_hbm.at[rows(own), :], copy_sems.at[2])
            out_cp.start()

            srcs = [send_buf, ag_recv.at[0], ag_recv.at[1]]
            for s in range(N - 1):
                slot = s % 2
                rdma = pltpu.make_async_remote_copy(
                    src_ref=srcs[s],
                    dst_ref=ag_recv.at[slot],
                    send_sem=send_sems.at[3 + s],
                    recv_sem=recv_sems.at[3 + s],
                    device_id=right,
                )
                if s == 2:
                    pl.semaphore_wait(credit_sem, 1)
                rdma.start()
                rdma.wait()
                if s == 1:
                    pl.semaphore_signal(credit_sem, 1, device_id=left)
                origin = (zi - s) % N
                load(ag_recv.at[slot], out_hbm.at[rows(origin), :],
                     copy_sems.at[3])
            out_cp.wait()

        for r in range(ROUNDS):
            one_round(r * RH)

    return pl.pallas_call(
        body,
        out_shape=jax.ShapeDtypeStruct((M, D), F32),
        in_specs=[
            pl.BlockSpec(memory_space=pltpu.MemorySpace.HBM),
            pl.BlockSpec(memory_space=pltpu.MemorySpace.HBM),
            pl.BlockSpec(memory_space=pltpu.MemorySpace.VMEM),
        ],
        out_specs=pl.BlockSpec(memory_space=pltpu.MemorySpace.HBM),
        scratch_shapes=[
            pltpu.VMEM((CH, D), F32),
            pltpu.VMEM((CH, D), F32),
            pltpu.VMEM((2, CH, D), F32),
            pltpu.VMEM((2, CH, D), F32),
            pltpu.SemaphoreType.DMA((6,)),
            pltpu.SemaphoreType.DMA((6,)),
            pltpu.SemaphoreType.DMA((4,)),
            pltpu.SemaphoreType.REGULAR,
        ],
        compiler_params=pltpu.CompilerParams(
            collective_id=0, vmem_limit_bytes=60 * 1024 * 1024),
    )(partial, resid, gamma2)


# device time: 1186839 ns/iter; 1.0232x vs baseline; 1.0232x over previous
import jax
import jax.numpy as jnp
from jax import lax
from jax.experimental import pallas as pl
from jax.experimental.pallas import tpu as pltpu

N = 4
M = 4096
D = 4096
ROUNDS = 2
RH = M // ROUNDS
CH = RH // N
CH2 = CH // 2
TR = 128
EPS = 1e-6
F32 = jnp.float32


def kernel(partial, resid, gamma):
    gamma2 = gamma.reshape(1, D)

    def body(x_hbm, resid_hbm, gamma_ref, out_hbm,
             pbuf, send_buf, rs_recv, ag_recv,
             send_sems, recv_sems, copy_sems, out_sems, credit_sems):
        xi = lax.axis_index("x")
        yi = lax.axis_index("y")
        zi = lax.axis_index("z")
        right = (xi, yi, (zi + 1) % N)
        left = (xi, yi, (zi - 1) % N)

        bar = pltpu.get_barrier_semaphore()
        pl.semaphore_signal(bar, 1, device_id=left)
        pl.semaphore_signal(bar, 1, device_id=right)
        pl.semaphore_wait(bar, 2)

        def copy(src, dst, sem):
            cp = pltpu.make_async_copy(src, dst, sem)
            cp.start()
            return cp

        def tiled(f):
            lax.fori_loop(0, CH2 // TR,
                          lambda t, _: (f(pl.ds(t * TR, TR)), 0)[1], 0)

        DIRS = (
            dict(i=0, tgt=right, csrc=left, off=0,
                 step=lambda s: (zi - s) % N, own=(zi + 1) % N),
            dict(i=1, tgt=left, csrc=right, off=CH2,
                 step=lambda s: (zi + s) % N, own=(zi - 1) % N),
        )

        def one_round(base):
            def rows(d, c):
                return pl.ds(base + c * CH + d["off"], CH2)

            cps = []
            for d in DIRS:
                i = d["i"]
                cps.append(copy(x_hbm.at[0, rows(d, d["step"](0)), :],
                                send_buf.at[i], copy_sems.at[i]))
                cps.append(copy(x_hbm.at[0, rows(d, d["step"](1)), :],
                                pbuf.at[i], copy_sems.at[2 + i]))
            for cp in cps:
                cp.wait()

            pload = [None, None]
            for s in range(N - 1):
                slot = s % 2
                rdmas = []
                for d in DIRS:
                    i = d["i"]
                    rdma = pltpu.make_async_remote_copy(
                        src_ref=send_buf.at[i],
                        dst_ref=rs_recv.at[i, slot],
                        send_sem=send_sems.at[i, s],
                        recv_sem=recv_sems.at[i, s],
                        device_id=d["tgt"],
                    )
                    if s == 2:
                        pl.semaphore_wait(credit_sems.at[i], 1)
                    rdma.start()
                    rdmas.append(rdma)
                for d, rdma in zip(DIRS, rdmas):
                    i = d["i"]
                    rdma.wait()
                    if pload[i] is not None:
                        pload[i].wait()

                    def add(ts, i=i, slot=slot):
                        send_buf[i, ts, :] = (
                            rs_recv[i, slot, ts, :] + pbuf[i, ts, :])
                    tiled(add)
                    if s == 0:
                        pl.semaphore_signal(credit_sems.at[i], 1,
                                            device_id=d["csrc"])
                    if s < 2:
                        pload[i] = copy(
                            x_hbm.at[0, rows(d, d["step"](s + 2)), :],
                            pbuf.at[i], copy_sems.at[2 + i])

            own_cps = []
            for d in DIRS:
                i = d["i"]
                copy(resid_hbm.at[rows(d, d["own"]), :], pbuf.at[i],
                     copy_sems.at[i]).wait()

                def norm(ts, i=i):
                    y = send_buf[i, ts, :] + pbuf[i, ts, :]
                    ms = jnp.mean(y * y, axis=-1, keepdims=True)
                    send_buf[i, ts, :] = (
                        (y * lax.rsqrt(ms + EPS)) * gamma_ref[...])
                tiled(norm)
                own_cps.append(copy(send_buf.at[i],
                                    out_hbm.at[rows(d, d["own"]), :],
                                    copy_sems.at[2 + i]))

            out_cp = [[None, None], [None, None]]
            for s in range(N - 1):
                slot = s % 2
                rdmas = []
                for d in DIRS:
                    i = d["i"]
                    src = send_buf.at[i] if s == 0 else ag_recv.at[i, s - 1]
                    rdma = pltpu.make_async_remote_copy(
                        src_ref=src,
                        dst_ref=ag_recv.at[i, slot],
                        send_sem=send_sems.at[i, 3 + s],
                        recv_sem=recv_sems.at[i, 3 + s],
                        device_id=d["tgt"],
                    )
                    if s == 2:
                        pl.semaphore_wait(credit_sems.at[i], 1)
                    rdma.start()
                    rdmas.append(rdma)
                for d, rdma in zip(DIRS, rdmas):
                    i = d["i"]
                    rdma.wait()
                    if s == 1:
                        out_cp[i][0].wait()
                        pl.semaphore_signal(credit_sems.at[i], 1,
                                            device_id=d["csrc"])
                    org = (zi - s) % N if i == 0 else (zi + s) % N
                    out_cp[i][slot] = copy(
                        ag_recv.at[i, slot], out_hbm.at[rows(d, org), :],
                        out_sems.at[i, slot])
            for cps in out_cp:
                for cp in cps:
                    cp.wait()
            for cp in own_cps:
                cp.wait()

        for r in range(ROUNDS):
            one_round(r * RH)

    return pl.pallas_call(
        body,
        out_shape=jax.ShapeDtypeStruct((M, D), F32),
        in_specs=[
            pl.BlockSpec(memory_space=pltpu.MemorySpace.HBM),
            pl.BlockSpec(memory_space=pltpu.MemorySpace.HBM),
            pl.BlockSpec(memory_space=pltpu.MemorySpace.VMEM),
        ],
        out_specs=pl.BlockSpec(memory_space=pltpu.MemorySpace.HBM),
        scratch_shapes=[
            pltpu.VMEM((2, CH2, D), F32),
            pltpu.VMEM((2, CH2, D), F32),
            pltpu.VMEM((2, 2, CH2, D), F32),
            pltpu.VMEM((2, 2, CH2, D), F32),
            pltpu.SemaphoreType.DMA((2, 6)),
            pltpu.SemaphoreType.DMA((2, 6)),
            pltpu.SemaphoreType.DMA((4,)),
            pltpu.SemaphoreType.DMA((2, 2)),
            pltpu.SemaphoreType.REGULAR((2,)),
        ],
        compiler_params=pltpu.CompilerParams(
            collective_id=0, vmem_limit_bytes=60 * 1024 * 1024),
    )(partial, resid, gamma2)
